# baseline (speedup 1.0000x reference)
import functools

import jax
import jax.numpy as jnp
from jax import lax
from jax.experimental import pallas as pl
from jax.experimental.pallas import tpu as pltpu
from jax.experimental.pallas import tpu_sc as plsc


def _make_sc_loss(B, H, W, n_pad):
    HW = H * W
    info = plsc.get_sparse_core_info()
    NC, NS, L = 1, info.num_subcores, info.num_lanes
    SPT = n_pad // NS
    CB = B // NC
    NPH = CB // 2
    PB = CB // NPH
    RPT = PB * H // NS
    NBLK = RPT // 8
    mesh = plsc.VectorSubcoreMesh(
        core_axis_name="c", subcore_axis_name="s", num_cores=NC)

    @functools.partial(
        pl.kernel,
        out_type=jax.ShapeDtypeStruct((NC * NS, L), jnp.float32),
        mesh=mesh,
        scratch_types=[
            pltpu.VMEM_SHARED((PB * HW,), jnp.float32),
            pltpu.VMEM((RPT // 2, W), jnp.float32),
            pltpu.VMEM((RPT // 2, W), jnp.float32),
            pltpu.VMEM((SPT,), jnp.int32),
            pltpu.VMEM((SPT,), jnp.int32),
            pltpu.VMEM((SPT,), jnp.int32),
            pltpu.VMEM((PB, 128), jnp.int32),
            pltpu.VMEM((PB, 128), jnp.float32),
            pltpu.VMEM((CB * SPT,), jnp.float32),
            pltpu.VMEM((SPT,), jnp.float32),
            pltpu.VMEM((L,), jnp.float32),
            pltpu.SemaphoreType.DMA,
            pltpu.SemaphoreType.DMA,
            pltpu.SemaphoreType.DMA,
        ],
    )
    def sc_loss(pred_hbm, tgt_hbm, rows_hbm, cols_hbm, mask_hbm, parts_hbm,
                sp, tspa, tspb, rows_v, cols_v, sidx_v, fidx_v, g_v, tgt_v,
                mask_v, part_v, semc, semr, semg):
        cid = lax.axis_index("c")
        sid = lax.axis_index("s")
        wid = sid * NC + cid

        pltpu.sync_copy(rows_hbm.at[pl.ds(sid * SPT, SPT)], rows_v)
        pltpu.sync_copy(cols_hbm.at[pl.ds(sid * SPT, SPT)], cols_v)
        pltpu.sync_copy(mask_hbm.at[pl.ds(sid * SPT, SPT)], mask_v)
        pltpu.sync_copy(
            tgt_hbm.at[pl.ds(sid * (B * SPT) + cid * (CB * SPT), CB * SPT)],
            tgt_v)

        for c in range(SPT // L):
            r = rows_v[pl.ds(c * L, L)]
            cc = cols_v[pl.ds(c * L, L)]
            sidx_v[pl.ds(c * L, L)] = r * W + cc
        for li in range(PB):
            for c in range(SPT // L):
                fidx_v[li, pl.ds(c * L, L)] = \
                    sidx_v[pl.ds(c * L, L)] + li * HW

        tsp = [tspa, tspb]
        HB = RPT // 2

        def block_copy(p, i):
            base_row = (cid * CB + p * PB) * H + sid * RPT
            g8 = pl.multiple_of(base_row + i * HB, 8)
            return pltpu.make_async_copy(
                pred_hbm.at[pl.ds(g8, HB), :], tsp[i], semc)

        def start_blocks(p):
            for i in range(2):
                block_copy(p, i).start()

        def forward_phase(p):
            for i in range(2):
                block_copy(p, i).wait()
                sp_base = sid * RPT + i * HB
                for rr in range(HB):
                    pltpu.make_async_copy(
                        tsp[i].at[rr, :],
                        sp.at[pl.ds((sp_base + rr) * W, W)], semr).start()
            for i in range(2):
                pltpu.make_async_copy(
                    pred_hbm.at[pl.ds(0, HB), :], tsp[i], semr).wait()

        acc = jnp.zeros((L,), jnp.float32)
        start_blocks(0)
        for p in range(NPH):
            forward_phase(p)
            if p + 1 < NPH:
                start_blocks(p + 1)
            plsc.subcore_barrier()
            gathers = [
                pltpu.make_async_copy(sp.at[fidx_v.at[li]], g_v.at[li], semg)
                for li in range(PB)
            ]
            for cp in gathers:
                cp.start()
            for cp in gathers:
                cp.wait()
            for li in range(PB):
                bl = p * PB + li
                for c in range(SPT // L):
                    g = g_v[li, pl.ds(c * L, L)]
                    t = tgt_v[pl.ds(bl * SPT + c * L, L)]
                    acc = acc + jnp.abs(g - t) * mask_v[pl.ds(c * L, L)]
            plsc.subcore_barrier()

        part_v[...] = acc
        pltpu.sync_copy(part_v, parts_hbm.at[wid])

    return sc_loss


def _reduce_body(scale, parts_ref, out_ref):
    out_ref[...] = (jnp.sum(parts_ref[...]) * scale)[None, None]


def kernel(pred_images, target_runoff_values, station_rows, station_cols):
    B, _, H, W = pred_images.shape
    N = station_rows.shape[0]
    NS = 16
    SPT = -(-N // NS)
    SPT = -(-SPT // 128) * 128
    n_pad = SPT * NS

    pred2 = pred_images.reshape(B * H, W)
    rows_p = jnp.pad(station_rows, (0, n_pad - N))
    cols_p = jnp.pad(station_cols, (0, n_pad - N))
    tgt_p = jnp.pad(target_runoff_values[:, :B], ((0, n_pad - N), (0, 0)))
    tgt_prep = tgt_p.reshape(NS, SPT, B).transpose(0, 2, 1).reshape(-1)
    mask = (jnp.arange(n_pad, dtype=jnp.int32) < N).astype(jnp.float32)

    parts = _make_sc_loss(B, H, W, n_pad)(
        pred2, tgt_prep, rows_p, cols_p, mask)

    out = pl.pallas_call(
        functools.partial(_reduce_body, 1.0 / (B * N)),
        out_shape=jax.ShapeDtypeStruct((1, 1), jnp.float32),
    )(parts)
    return out[0, 0]


# speedup vs baseline: 1.0038x; 1.0038x over previous
import functools

import jax
import jax.numpy as jnp
from jax import lax
from jax.experimental import pallas as pl
from jax.experimental.pallas import tpu as pltpu
from jax.experimental.pallas import tpu_sc as plsc


def _make_sc_loss(B, HW, W, n_pad):
    info = plsc.get_sparse_core_info()
    NC, NS, L = info.num_cores, info.num_subcores, info.num_lanes
    NW = NC * NS
    SPT = n_pad // NW
    E = SPT * B
    NROW = E // 128
    mesh = plsc.VectorSubcoreMesh(core_axis_name="c", subcore_axis_name="s")

    @functools.partial(
        pl.kernel,
        out_type=jax.ShapeDtypeStruct((NW, L), jnp.float32),
        mesh=mesh,
        scratch_types=[
            pltpu.VMEM((SPT,), jnp.int32),
            pltpu.VMEM((SPT,), jnp.int32),
            pltpu.VMEM((SPT,), jnp.int32),
            pltpu.VMEM((NROW, 128), jnp.int32),
            pltpu.VMEM((NROW, 128), jnp.float32),
            pltpu.VMEM((E,), jnp.float32),
            pltpu.VMEM((SPT,), jnp.float32),
            pltpu.VMEM((L,), jnp.float32),
            pltpu.SemaphoreType.DMA,
        ],
    )
    def sc_loss(pred_hbm, tgt_hbm, rows_hbm, cols_hbm, mask_hbm, parts_hbm,
                rows_v, cols_v, sidx_v, fidx_v, g_v, tgt_v, mask_v, part_v,
                sem):
        wid = lax.axis_index("s") * NC + lax.axis_index("c")
        base_s = wid * SPT
        pltpu.sync_copy(rows_hbm.at[pl.ds(base_s, SPT)], rows_v)
        pltpu.sync_copy(cols_hbm.at[pl.ds(base_s, SPT)], cols_v)
        pltpu.sync_copy(mask_hbm.at[pl.ds(base_s, SPT)], mask_v)
        pltpu.sync_copy(tgt_hbm.at[pl.ds(wid * E, E)], tgt_v)

        for c in range(SPT // L):
            r = rows_v[pl.ds(c * L, L)]
            cc = cols_v[pl.ds(c * L, L)]
            sidx_v[pl.ds(c * L, L)] = r * W + cc

        for b in range(B):
            for c in range(SPT // L):
                o = b * SPT + c * L
                fidx_v[o // 128, pl.ds(o % 128, L)] = \
                    sidx_v[pl.ds(c * L, L)] + b * HW

        copies = [
            pltpu.make_async_copy(pred_hbm.at[fidx_v.at[j]], g_v.at[j], sem)
            for j in range(NROW)
        ]
        for cp in copies:
            cp.start()
        for cp in copies:
            cp.wait()

        acc = jnp.zeros((L,), jnp.float32)
        for b in range(B):
            for c in range(SPT // L):
                o = b * SPT + c * L
                g = g_v[o // 128, pl.ds(o % 128, L)]
                t = tgt_v[pl.ds(b * SPT + c * L, L)]
                acc = acc + jnp.abs(g - t) * mask_v[pl.ds(c * L, L)]
        part_v[...] = acc
        pltpu.sync_copy(part_v, parts_hbm.at[wid])

    return sc_loss


def _reduce_body(scale, parts_ref, out_ref):
    out_ref[...] = (jnp.sum(parts_ref[...]) * scale)[None, None]


def kernel(pred_images, target_runoff_values, station_rows, station_cols):
    B, _, H, W = pred_images.shape
    N = station_rows.shape[0]
    HW = H * W
    NW = 32
    SPT = -(-N // NW)
    SPT = -(-SPT // 8) * 8
    n_pad = SPT * NW

    one = target_runoff_values[0, 0] * 0.0 + 1.0
    pred_flat = pred_images.reshape(-1) * one
    rows_p = jnp.pad(station_rows, (0, n_pad - N))
    cols_p = jnp.pad(station_cols, (0, n_pad - N))
    SPT_ = n_pad // NW
    tgt_p = jnp.pad(target_runoff_values[:, :B], ((0, n_pad - N), (0, 0)))
    tgt_prep = tgt_p.reshape(NW, SPT_, B).transpose(0, 2, 1).reshape(-1)
    mask = (jnp.arange(n_pad, dtype=jnp.int32) < N).astype(jnp.float32)

    parts = _make_sc_loss(B, HW, W, n_pad)(
        pred_flat, tgt_prep, rows_p, cols_p, mask)

    out = pl.pallas_call(
        functools.partial(_reduce_body, 1.0 / (B * N)),
        out_shape=jax.ShapeDtypeStruct((1, 1), jnp.float32),
    )(parts)
    return out[0, 0]


# speedup vs baseline: 1.3700x; 1.3649x over previous
import functools

import jax
import jax.numpy as jnp
from jax import lax
from jax.experimental import pallas as pl
from jax.experimental.pallas import tpu as pltpu
from jax.experimental.pallas import tpu_sc as plsc


def _make_sc_loss(B, H, W, n_pad):
    HW = H * W
    info = plsc.get_sparse_core_info()
    NC, NS, L = info.num_cores, info.num_subcores, info.num_lanes
    SPT = n_pad // NS
    CB = B // NC
    NPH = CB // 2
    PB = CB // NPH
    RPT = PB * H // NS
    NBLK = RPT // 8
    mesh = plsc.VectorSubcoreMesh(
        core_axis_name="c", subcore_axis_name="s", num_cores=NC)

    @functools.partial(
        pl.kernel,
        out_type=jax.ShapeDtypeStruct((NC * NS, L), jnp.float32),
        mesh=mesh,
        scratch_types=[
            pltpu.VMEM_SHARED((PB * HW,), jnp.float32),
            pltpu.VMEM((RPT // 2, W), jnp.float32),
            pltpu.VMEM((RPT // 2, W), jnp.float32),
            pltpu.VMEM((SPT,), jnp.int32),
            pltpu.VMEM((SPT,), jnp.int32),
            pltpu.VMEM((SPT,), jnp.int32),
            pltpu.VMEM((PB, 128), jnp.int32),
            pltpu.VMEM((PB, 128), jnp.float32),
            pltpu.VMEM((CB * SPT,), jnp.float32),
            pltpu.VMEM((SPT,), jnp.float32),
            pltpu.VMEM((L,), jnp.float32),
            pltpu.SemaphoreType.DMA,
            pltpu.SemaphoreType.DMA,
            pltpu.SemaphoreType.DMA,
        ],
    )
    def sc_loss(pred_hbm, tgt_hbm, rows_hbm, cols_hbm, mask_hbm, parts_hbm,
                sp, tspa, tspb, rows_v, cols_v, sidx_v, fidx_v, g_v, tgt_v,
                mask_v, part_v, semc, semr, semg):
        cid = lax.axis_index("c")
        sid = lax.axis_index("s")
        wid = sid * NC + cid

        tsp = [tspa, tspb]
        HB = RPT // 2

        def block_copy(p, i):
            base_row = (cid * CB + p * PB) * H + sid * RPT
            g8 = pl.multiple_of(base_row + i * HB, 8)
            return pltpu.make_async_copy(
                pred_hbm.at[pl.ds(g8, HB), :], tsp[i], semc)

        def start_blocks(p):
            for i in range(2):
                block_copy(p, i).start()

        def forward_phase(p):
            for i in range(2):
                block_copy(p, i).wait()
                sp_base = sid * RPT + i * HB
                for rr in range(HB):
                    pltpu.make_async_copy(
                        tsp[i].at[rr, :],
                        sp.at[pl.ds((sp_base + rr) * W, W)], semr).start()
            for i in range(2):
                pltpu.make_async_copy(
                    pred_hbm.at[pl.ds(0, HB), :], tsp[i], semr).wait()

        acc = jnp.zeros((L,), jnp.float32)
        start_blocks(0)

        pro = [
            pltpu.make_async_copy(
                rows_hbm.at[pl.ds(sid * SPT, SPT)], rows_v, semg),
            pltpu.make_async_copy(
                cols_hbm.at[pl.ds(sid * SPT, SPT)], cols_v, semg),
            pltpu.make_async_copy(
                mask_hbm.at[pl.ds(sid * SPT, SPT)], mask_v, semg),
            pltpu.make_async_copy(
                tgt_hbm.at[pl.ds(sid * (B * SPT) + cid * (CB * SPT),
                                 CB * SPT)], tgt_v, semg),
        ]
        for cp in pro:
            cp.start()
        pro[0].wait()
        pro[1].wait()
        for c in range(SPT // L):
            r = rows_v[pl.ds(c * L, L)]
            cc = cols_v[pl.ds(c * L, L)]
            sidx_v[pl.ds(c * L, L)] = r * W + cc
        for li in range(PB):
            for c in range(SPT // L):
                fidx_v[li, pl.ds(c * L, L)] = \
                    sidx_v[pl.ds(c * L, L)] + li * HW
        pro[2].wait()
        pro[3].wait()

        for p in range(NPH):
            forward_phase(p)
            if p + 1 < NPH:
                start_blocks(p + 1)
            plsc.subcore_barrier()
            gathers = [
                pltpu.make_async_copy(sp.at[fidx_v.at[li]], g_v.at[li], semg)
                for li in range(PB)
            ]
            for cp in gathers:
                cp.start()
            for cp in gathers:
                cp.wait()
            for li in range(PB):
                bl = p * PB + li
                for c in range(SPT // L):
                    g = g_v[li, pl.ds(c * L, L)]
                    t = tgt_v[pl.ds(bl * SPT + c * L, L)]
                    acc = acc + jnp.abs(g - t) * mask_v[pl.ds(c * L, L)]
            plsc.subcore_barrier()

        part_v[...] = acc
        pltpu.sync_copy(part_v, parts_hbm.at[wid])

    return sc_loss


def _reduce_body(scale, parts_ref, out_ref):
    out_ref[...] = (jnp.sum(parts_ref[...]) * scale)[None, None]


def kernel(pred_images, target_runoff_values, station_rows, station_cols):
    B, _, H, W = pred_images.shape
    N = station_rows.shape[0]
    NS = 16
    SPT = -(-N // NS)
    SPT = -(-SPT // 128) * 128
    n_pad = SPT * NS

    pred2 = pred_images.reshape(B * H, W)
    rows_p = jnp.pad(station_rows, (0, n_pad - N))
    cols_p = jnp.pad(station_cols, (0, n_pad - N))
    tgt_p = jnp.pad(target_runoff_values[:, :B], ((0, n_pad - N), (0, 0)))
    tgt_prep = tgt_p.reshape(NS, SPT, B).transpose(0, 2, 1).reshape(-1)
    mask = (jnp.arange(n_pad, dtype=jnp.int32) < N).astype(jnp.float32)

    parts = _make_sc_loss(B, H, W, n_pad)(
        pred2, tgt_prep, rows_p, cols_p, mask)

    out = pl.pallas_call(
        functools.partial(_reduce_body, 1.0 / (B * N)),
        out_shape=jax.ShapeDtypeStruct((1, 1), jnp.float32),
    )(parts)
    return out[0, 0]


# speedup vs baseline: 1.6106x; 1.1756x over previous
import functools

import jax
import jax.numpy as jnp
from jax import lax
from jax.experimental import pallas as pl
from jax.experimental.pallas import tpu as pltpu
from jax.experimental.pallas import tpu_sc as plsc


def _make_sc_loss(B, H, W, N, n_pad):
    HW = H * W
    info = plsc.get_sparse_core_info()
    NC, NS, L = info.num_cores, info.num_subcores, info.num_lanes
    SPT = n_pad // NS
    CB = B // NC
    NPH = CB // 2
    PB = CB // NPH
    RPT = PB * H // NS
    HB = RPT // 2
    tail_sid = N // SPT
    TAIL = N - tail_sid * SPT
    mesh = plsc.VectorSubcoreMesh(
        core_axis_name="c", subcore_axis_name="s", num_cores=NC)

    @functools.partial(
        pl.kernel,
        out_type=jax.ShapeDtypeStruct((NC * NS, L), jnp.float32),
        mesh=mesh,
        scratch_types=[
            pltpu.VMEM_SHARED((PB * HW,), jnp.float32),
            pltpu.VMEM((HB, W), jnp.float32),
            pltpu.VMEM((HB, W), jnp.float32),
            pltpu.VMEM((SPT,), jnp.int32),
            pltpu.VMEM((SPT,), jnp.int32),
            pltpu.VMEM((SPT,), jnp.int32),
            pltpu.VMEM((PB, 128), jnp.int32),
            pltpu.VMEM((PB, 128), jnp.float32),
            pltpu.VMEM((CB * SPT,), jnp.float32),
            pltpu.VMEM((L,), jnp.float32),
            pltpu.SemaphoreType.DMA,
            pltpu.SemaphoreType.DMA,
            pltpu.SemaphoreType.DMA,
        ],
    )
    def sc_loss(pred_hbm, tgt_hbm, rows_hbm, cols_hbm, parts_hbm,
                sp, tspa, tspb, rows_v, cols_v, sidx_v, fidx_v, g_v, tgt_v,
                part_v, semc, semr, semg):
        cid = lax.axis_index("c")
        sid = lax.axis_index("s")
        wid = sid * NC + cid
        base_s = sid * SPT

        tsp = [tspa, tspb]

        def block_copy(p, i):
            base_row = (cid * CB + p * PB) * H + sid * RPT
            g8 = pl.multiple_of(base_row + i * HB, 8)
            return pltpu.make_async_copy(
                pred_hbm.at[pl.ds(g8, HB), :], tsp[i], semc)

        def start_blocks(p):
            for i in range(2):
                block_copy(p, i).start()

        start_blocks(0)

        pro = []

        @pl.when(sid != tail_sid)
        def _():
            pltpu.make_async_copy(
                rows_hbm.at[pl.ds(base_s, SPT)], rows_v, semg).start()
            pltpu.make_async_copy(
                cols_hbm.at[pl.ds(base_s, SPT)], cols_v, semg).start()

        if TAIL:
            @pl.when(sid == tail_sid)
            def _():
                pltpu.make_async_copy(
                    rows_hbm.at[pl.ds(base_s, TAIL)],
                    rows_v.at[pl.ds(0, TAIL)], semg).start()
                pltpu.make_async_copy(
                    cols_hbm.at[pl.ds(base_s, TAIL)],
                    cols_v.at[pl.ds(0, TAIL)], semg).start()

        cp = pltpu.make_async_copy(
            tgt_hbm.at[pl.ds(sid * (B * SPT) + cid * (CB * SPT), CB * SPT)],
            tgt_v, semg)
        cp.start()
        pro.append(cp)

        @pl.when(sid != tail_sid)
        def _():
            pltpu.make_async_copy(
                rows_hbm.at[pl.ds(0, SPT)], rows_v, semg).wait()
            pltpu.make_async_copy(
                cols_hbm.at[pl.ds(0, SPT)], cols_v, semg).wait()

        if TAIL:
            @pl.when(sid == tail_sid)
            def _():
                pltpu.make_async_copy(
                    rows_hbm.at[pl.ds(0, TAIL)],
                    rows_v.at[pl.ds(0, TAIL)], semg).wait()
                pltpu.make_async_copy(
                    cols_hbm.at[pl.ds(0, TAIL)],
                    cols_v.at[pl.ds(0, TAIL)], semg).wait()

        for c in range(SPT // L):
            r = rows_v[pl.ds(c * L, L)]
            cc = cols_v[pl.ds(c * L, L)]
            off = r * W + cc
            off = jnp.minimum(jnp.maximum(off, 0), HW - 1)
            sidx_v[pl.ds(c * L, L)] = off
        for li in range(PB):
            for c in range(SPT // L):
                fidx_v[li, pl.ds(c * L, L)] = \
                    sidx_v[pl.ds(c * L, L)] + li * HW

        iota = lax.iota(jnp.int32, L)
        masks = []
        for c in range(SPT // L):
            g = base_s + c * L + iota
            m = jnp.minimum(jnp.maximum(N - g, 0), 1)
            masks.append(m.astype(jnp.float32))

        for cp in pro:
            cp.wait()

        def forward_phase(p):
            for i in range(2):
                block_copy(p, i).wait()
                sp_base = sid * RPT + i * HB
                buf = tsp[i]

                def fwd(rr, carry):
                    pltpu.make_async_copy(
                        buf.at[rr, :],
                        sp.at[pl.ds((sp_base + rr) * W, W)], semr).start()
                    return carry

                lax.fori_loop(0, HB, fwd, 0)
            for i in range(2):
                pltpu.make_async_copy(
                    pred_hbm.at[pl.ds(0, HB), :], tsp[i], semr).wait()

        acc = jnp.zeros((L,), jnp.float32)
        for p in range(NPH):
            forward_phase(p)
            if p + 1 < NPH:
                start_blocks(p + 1)
            plsc.subcore_barrier()
            gathers = [
                pltpu.make_async_copy(sp.at[fidx_v.at[li]], g_v.at[li], semg)
                for li in range(PB)
            ]
            for cp in gathers:
                cp.start()
            for cp in gathers:
                cp.wait()
            for li in range(PB):
                bl = p * PB + li
                for c in range(SPT // L):
                    g = g_v[li, pl.ds(c * L, L)]
                    t = tgt_v[pl.ds(bl * SPT + c * L, L)]
                    acc = acc + jnp.abs(g - t) * masks[c]
            plsc.subcore_barrier()

        part_v[...] = acc
        pltpu.sync_copy(part_v, parts_hbm.at[wid])

    return sc_loss


def _reduce_body(scale, parts_ref, out_ref):
    out_ref[...] = (jnp.sum(parts_ref[...]) * scale)[None, None]


def kernel(pred_images, target_runoff_values, station_rows, station_cols):
    B, _, H, W = pred_images.shape
    N = station_rows.shape[0]
    NS = 16
    SPT = -(-N // NS)
    SPT = -(-SPT // 128) * 128
    n_pad = SPT * NS

    pred2 = pred_images.reshape(B * H, W)
    tgt_p = jnp.pad(target_runoff_values[:, :B], ((0, n_pad - N), (0, 0)))
    tgt_prep = tgt_p.reshape(NS, SPT, B).transpose(0, 2, 1).reshape(-1)

    parts = _make_sc_loss(B, H, W, N, n_pad)(
        pred2, tgt_prep, station_rows, station_cols)

    out = pl.pallas_call(
        functools.partial(_reduce_body, 1.0 / (B * N)),
        out_shape=jax.ShapeDtypeStruct((1, 1), jnp.float32),
    )(parts)
    return out[0, 0]


# speedup vs baseline: 1.6114x; 1.0005x over previous
import functools

import jax
import jax.numpy as jnp
from jax import lax
from jax.experimental import pallas as pl
from jax.experimental.pallas import tpu as pltpu
from jax.experimental.pallas import tpu_sc as plsc


def _make_sc_loss(B, H, W, N, n_pad):
    HW = H * W
    info = plsc.get_sparse_core_info()
    NC, NS, L = info.num_cores, info.num_subcores, info.num_lanes
    SPT = n_pad // NS
    CB = B // NC
    NPH = CB // 2
    PB = CB // NPH
    RPT = PB * H // NS
    HB = RPT // 2
    tail_sid = N // SPT
    TAIL = N - tail_sid * SPT
    mesh = plsc.VectorSubcoreMesh(
        core_axis_name="c", subcore_axis_name="s", num_cores=NC)

    @functools.partial(
        pl.kernel,
        out_type=jax.ShapeDtypeStruct((NC * NS, L), jnp.float32),
        mesh=mesh,
        scratch_types=[
            pltpu.VMEM_SHARED((PB * HW,), jnp.float32),
            pltpu.VMEM((HB, W), jnp.float32),
            pltpu.VMEM((HB, W), jnp.float32),
            pltpu.VMEM((SPT,), jnp.int32),
            pltpu.VMEM((SPT,), jnp.int32),
            pltpu.VMEM((SPT,), jnp.int32),
            pltpu.VMEM((PB, 128), jnp.int32),
            pltpu.VMEM((PB, 128), jnp.float32),
            pltpu.VMEM((CB * SPT,), jnp.float32),
            pltpu.VMEM((SPT,), jnp.float32),
            pltpu.VMEM((L,), jnp.float32),
            pltpu.SemaphoreType.DMA,
            pltpu.SemaphoreType.DMA,
            pltpu.SemaphoreType.DMA,
        ],
    )
    def sc_loss(pred_hbm, tgt_hbm, rows_hbm, cols_hbm, parts_hbm,
                sp, tspa, tspb, rows_v, cols_v, sidx_v, fidx_v, g_v, tgt_v,
                mask_v, part_v, semc, semr, semg):
        cid = lax.axis_index("c")
        sid = lax.axis_index("s")
        wid = sid * NC + cid
        base_s = sid * SPT

        tsp = [tspa, tspb]

        def block_copy(p, i):
            base_row = (cid * CB + p * PB) * H + sid * RPT
            g8 = pl.multiple_of(base_row + i * HB, 8)
            return pltpu.make_async_copy(
                pred_hbm.at[pl.ds(g8, HB), :], tsp[i], semc)

        def start_blocks(p):
            for i in range(2):
                block_copy(p, i).start()

        start_blocks(0)

        pro = []

        @pl.when(sid != tail_sid)
        def _():
            pltpu.make_async_copy(
                rows_hbm.at[pl.ds(base_s, SPT)], rows_v, semg).start()
            pltpu.make_async_copy(
                cols_hbm.at[pl.ds(base_s, SPT)], cols_v, semg).start()

        if TAIL:
            @pl.when(sid == tail_sid)
            def _():
                pltpu.make_async_copy(
                    rows_hbm.at[pl.ds(base_s, TAIL)],
                    rows_v.at[pl.ds(0, TAIL)], semg).start()
                pltpu.make_async_copy(
                    cols_hbm.at[pl.ds(base_s, TAIL)],
                    cols_v.at[pl.ds(0, TAIL)], semg).start()

        cp = pltpu.make_async_copy(
            tgt_hbm.at[pl.ds(sid * (B * SPT) + cid * (CB * SPT), CB * SPT)],
            tgt_v, semg)
        cp.start()
        pro.append(cp)

        @pl.when(sid != tail_sid)
        def _():
            pltpu.make_async_copy(
                rows_hbm.at[pl.ds(0, SPT)], rows_v, semg).wait()
            pltpu.make_async_copy(
                cols_hbm.at[pl.ds(0, SPT)], cols_v, semg).wait()

        if TAIL:
            @pl.when(sid == tail_sid)
            def _():
                pltpu.make_async_copy(
                    rows_hbm.at[pl.ds(0, TAIL)],
                    rows_v.at[pl.ds(0, TAIL)], semg).wait()
                pltpu.make_async_copy(
                    cols_hbm.at[pl.ds(0, TAIL)],
                    cols_v.at[pl.ds(0, TAIL)], semg).wait()

        iota = lax.iota(jnp.int32, L)

        def build(c, carry):
            o = c * L
            r = rows_v[pl.ds(o, L)]
            cc = cols_v[pl.ds(o, L)]
            off = r * W + cc
            off = jnp.minimum(jnp.maximum(off, 0), HW - 1)
            sidx_v[pl.ds(o, L)] = off
            for li in range(PB):
                fidx_v[li, pl.ds(o, L)] = off + li * HW
            g = base_s + o + iota
            m = jnp.minimum(jnp.maximum(N - g, 0), 1)
            mask_v[pl.ds(o, L)] = m.astype(jnp.float32)
            return carry

        lax.fori_loop(0, SPT // L, build, 0)

        for cp in pro:
            cp.wait()

        def forward_phase(p):
            for i in range(2):
                block_copy(p, i).wait()
                sp_base = sid * RPT + i * HB
                buf = tsp[i]

                def fwd(rr, carry):
                    pltpu.make_async_copy(
                        buf.at[rr, :],
                        sp.at[pl.ds((sp_base + rr) * W, W)], semr).start()
                    return carry

                lax.fori_loop(0, HB, fwd, 0)
            for i in range(2):
                pltpu.make_async_copy(
                    pred_hbm.at[pl.ds(0, HB), :], tsp[i], semr).wait()

        acc = jnp.zeros((L,), jnp.float32)
        for p in range(NPH):
            forward_phase(p)
            if p + 1 < NPH:
                start_blocks(p + 1)
            plsc.subcore_barrier()
            gathers = [
                pltpu.make_async_copy(sp.at[fidx_v.at[li]], g_v.at[li], semg)
                for li in range(PB)
            ]
            for cp in gathers:
                cp.start()
            for cp in gathers:
                cp.wait()
            for li in range(PB):
                bl = p * PB + li

                def accum(c, a):
                    o = c * L
                    g = g_v[li, pl.ds(o, L)]
                    t = tgt_v[pl.ds(bl * SPT + o, L)]
                    return a + jnp.abs(g - t) * mask_v[pl.ds(o, L)]

                acc = lax.fori_loop(0, SPT // L, accum, acc)
            plsc.subcore_barrier()

        part_v[...] = acc
        pltpu.sync_copy(part_v, parts_hbm.at[wid])

    return sc_loss


def _reduce_body(scale, parts_ref, out_ref):
    out_ref[...] = (jnp.sum(parts_ref[...]) * scale)[None, None]


def kernel(pred_images, target_runoff_values, station_rows, station_cols):
    B, _, H, W = pred_images.shape
    N = station_rows.shape[0]
    NS = 16
    SPT = -(-N // NS)
    SPT = -(-SPT // 128) * 128
    n_pad = SPT * NS

    pred2 = pred_images.reshape(B * H, W)
    tgt_p = jnp.pad(target_runoff_values[:, :B], ((0, n_pad - N), (0, 0)))
    tgt_prep = tgt_p.reshape(NS, SPT, B).transpose(0, 2, 1).reshape(-1)

    parts = _make_sc_loss(B, H, W, N, n_pad)(
        pred2, tgt_prep, station_rows, station_cols)

    out = pl.pallas_call(
        functools.partial(_reduce_body, 1.0 / (B * N)),
        out_shape=jax.ShapeDtypeStruct((1, 1), jnp.float32),
    )(parts)
    return out[0, 0]
